# baseline (speedup 1.0000x reference)
import functools

import numpy as np
import jax
import jax.numpy as jnp
from jax.experimental import pallas as pl
from jax.experimental.pallas import tpu as pltpu

_i8 = np.arange(8)
_C8 = np.cos((2 * _i8[:, None] + 1) * _i8[None, :] * np.pi / 16)
_ALPHA = np.array([1.0 / np.sqrt(2)] + [1.0] * 7)
_D8 = (0.5 * _ALPHA[:, None] * _C8.T).astype(np.float32)

_RAW_Y = np.array(
    [[16, 11, 10, 16, 24, 40, 51, 61],
     [12, 12, 14, 19, 26, 58, 60, 55],
     [14, 13, 16, 24, 40, 57, 69, 56],
     [14, 17, 22, 29, 51, 87, 80, 62],
     [18, 22, 37, 56, 68, 109, 103, 77],
     [24, 35, 55, 64, 81, 104, 113, 92],
     [49, 64, 78, 87, 103, 121, 120, 101],
     [72, 92, 95, 98, 112, 100, 103, 99]], dtype=np.float32)
_RAW_C = np.full((8, 8), 99.0, dtype=np.float32)
_RAW_C[:4, :4] = np.array(
    [[17, 18, 24, 47],
     [18, 21, 26, 66],
     [24, 26, 56, 99],
     [47, 66, 99, 99]], dtype=np.float32).T
_GY = (0.5 * _RAW_Y).astype(np.float32)
_GC = (0.5 * _RAW_C.T).astype(np.float32)

_VMEM_LIMIT = 64 * 1024 * 1024


@functools.lru_cache(maxsize=None)
def _consts(R, W):
    Rc, Wc = R // 2, W // 2
    Gc = min(128, Wc)
    c = {}
    idct = np.einsum("ux,vy->uvyx", _D8, _D8).reshape(64, 64)
    c["mty"] = (_RAW_Y.T * 0.5).reshape(64, 1) * idct
    c["mtc"] = (_RAW_C * 0.5).reshape(64, 1) * idct
    c["pav"] = 0.5 * np.kron(np.eye(Rc, dtype=np.float32),
                             np.ones((1, 2), np.float32))
    c["urow"] = np.kron(np.eye(Rc, dtype=np.float32),
                        np.ones((2, 1), np.float32))
    c["hav"] = 0.5 * np.kron(np.eye(Gc, dtype=np.float32),
                             np.array([[1.0], [1.0]], np.float32))
    c["hup"] = np.kron(np.eye(Gc, dtype=np.float32),
                       np.array([[1.0, 1.0]], np.float32))
    return {k: jnp.asarray(np.asarray(v, np.float32)) for k, v in c.items()}


def _dot(a, b):
    return jnp.dot(a, b, preferred_element_type=jnp.float32)


def _lane_groups(t, mat, group):
    n = t.shape[1] // group
    if n == 1:
        return _dot(t, mat)
    return jnp.concatenate(
        [_dot(t[:, i * group:(i + 1) * group], mat) for i in range(n)], axis=1)


def _noise_img(rv_ref, m_ref, eps):
    a = rv_ref[...]
    wb_n, hb_n = a.shape[0], a.shape[1]
    n = _dot(a.reshape(wb_n * hb_n, 64), m_ref[...]) * eps
    return (n.reshape(wb_n, hb_n, 8, 8).transpose(1, 2, 0, 3)
            .reshape(hb_n * 8, wb_n * 8))


def _fused_kernel(x_ref, rv0_ref, rv1_ref, rv2_ref, eps_ref,
                  mty_ref, mtc_ref, pav_ref, urow_ref, hav_ref, hup_ref,
                  o_ref):
    eps = eps_ref[0, 0]
    xr = x_ref[0]
    xg = x_ref[1]
    xb = x_ref[2]
    y = 0.299 * xr + 0.587 * xg + 0.114 * xb
    cb = -0.168736 * xr - 0.331264 * xg + 0.5 * xb + 128.0
    cr = 0.5 * xr - 0.418688 * xg - 0.081312 * xb + 128.0

    Gc = hup_ref.shape[0]

    y_o = y + _noise_img(rv0_ref, mty_ref, eps)

    def chroma(rv_ref, plane):
        n = _noise_img(rv_ref, mtc_ref, eps)
        pd = _dot(pav_ref[...], plane)
        pd = _lane_groups(pd, hav_ref[...], 2 * Gc)
        s = pd + n
        u = _dot(urow_ref[...], s)
        return _lane_groups(u, hup_ref[...], Gc)

    cb_o = chroma(rv1_ref, cb)
    cr_o = chroma(rv2_ref, cr)

    cbm = cb_o - 128.0
    crm = cr_o - 128.0
    o_ref[0] = jnp.clip(y_o + 1.402 * crm, 0.0, 255.0)
    o_ref[1] = jnp.clip(y_o - 0.344136 * cbm - 0.714136 * crm, 0.0, 255.0)
    o_ref[2] = jnp.clip(y_o + 1.772 * cbm, 0.0, 255.0)


def kernel(x, rv0, rv1, rv2, epsilon):
    B, C, H, W = x.shape
    assert C == 3 and H % 16 == 0 and W % 16 == 0
    Hb, Wb = H // 8, W // 8
    Hc, Wc = H // 2, W // 2

    R = 256
    while H % R or R > H:
        R //= 2
    Rc = R // 2

    xf = x.astype(jnp.float32)
    eps2 = epsilon.astype(jnp.float32).reshape(B, 1, 1)
    rv0i = rv0.astype(jnp.float32).reshape(B, Wb, Hb, 64)
    rv1i = rv1.astype(jnp.float32).reshape(B, Wc // 8, Hc // 8, 64)
    rv2i = rv2.astype(jnp.float32).reshape(B, Wc // 8, Hc // 8, 64)

    c = _consts(R, W)
    grid = (B, H // R)

    def const_spec(a):
        return pl.BlockSpec(a.shape, lambda b, t: (0,) * a.ndim)

    out = pl.pallas_call(
        _fused_kernel,
        out_shape=jax.ShapeDtypeStruct((B, 3, H, W), jnp.float32),
        grid=grid,
        in_specs=[
            pl.BlockSpec((None, 3, R, W), lambda b, t: (b, 0, t, 0)),
            pl.BlockSpec((None, Wb, R // 8, 64), lambda b, t: (b, 0, t, 0)),
            pl.BlockSpec((None, Wc // 8, Rc // 8, 64), lambda b, t: (b, 0, t, 0)),
            pl.BlockSpec((None, Wc // 8, Rc // 8, 64), lambda b, t: (b, 0, t, 0)),
            pl.BlockSpec((None, 1, 1), lambda b, t: (b, 0, 0)),
            const_spec(c["mty"]),
            const_spec(c["mtc"]),
            const_spec(c["pav"]),
            const_spec(c["urow"]),
            const_spec(c["hav"]),
            const_spec(c["hup"]),
        ],
        out_specs=pl.BlockSpec((None, 3, R, W), lambda b, t: (b, 0, t, 0)),
        compiler_params=pltpu.CompilerParams(
            dimension_semantics=("parallel", "parallel"),
            vmem_limit_bytes=_VMEM_LIMIT),
    )(xf, rv0i, rv1i, rv2i, eps2,
      c["mty"], c["mtc"], c["pav"], c["urow"], c["hav"], c["hup"])
    return out


# speedup vs baseline: 1.0065x; 1.0065x over previous
import functools

import numpy as np
import jax
import jax.numpy as jnp
from jax.experimental import pallas as pl
from jax.experimental.pallas import tpu as pltpu

_i8 = np.arange(8)
_C8 = np.cos((2 * _i8[:, None] + 1) * _i8[None, :] * np.pi / 16)
_ALPHA = np.array([1.0 / np.sqrt(2)] + [1.0] * 7)
_D8 = (0.5 * _ALPHA[:, None] * _C8.T).astype(np.float32)

_RAW_Y = np.array(
    [[16, 11, 10, 16, 24, 40, 51, 61],
     [12, 12, 14, 19, 26, 58, 60, 55],
     [14, 13, 16, 24, 40, 57, 69, 56],
     [14, 17, 22, 29, 51, 87, 80, 62],
     [18, 22, 37, 56, 68, 109, 103, 77],
     [24, 35, 55, 64, 81, 104, 113, 92],
     [49, 64, 78, 87, 103, 121, 120, 101],
     [72, 92, 95, 98, 112, 100, 103, 99]], dtype=np.float32)
_RAW_C = np.full((8, 8), 99.0, dtype=np.float32)
_RAW_C[:4, :4] = np.array(
    [[17, 18, 24, 47],
     [18, 21, 26, 66],
     [24, 26, 56, 99],
     [47, 66, 99, 99]], dtype=np.float32).T
_VMEM_LIMIT = 64 * 1024 * 1024


@functools.lru_cache(maxsize=None)
def _consts(R, W):
    Rc, Wc = R // 2, W // 2
    Gc = min(128, Wc)
    c = {}
    idct = np.einsum("ux,vy->uvyx", _D8, _D8).reshape(64, 64)
    c["mty"] = (_RAW_Y.T * 0.5).reshape(64, 1) * idct
    c["mtc"] = (_RAW_C * 0.5).reshape(64, 1) * idct
    c["pav"] = 0.5 * np.kron(np.eye(Rc, dtype=np.float32),
                             np.ones((1, 2), np.float32))
    c["urow"] = np.kron(np.eye(Rc, dtype=np.float32),
                        np.ones((2, 1), np.float32))
    c["hav"] = 0.5 * np.kron(np.eye(Gc, dtype=np.float32),
                             np.array([[1.0], [1.0]], np.float32))
    c["hup"] = np.kron(np.eye(Gc, dtype=np.float32),
                       np.array([[1.0, 1.0]], np.float32))
    return {k: jnp.asarray(np.asarray(v, np.float32)) for k, v in c.items()}


def _dot(a, b):
    return jnp.dot(a, b, preferred_element_type=jnp.float32)


def _lane_groups(t, mat, group):
    n = t.shape[1] // group
    if n == 1:
        return _dot(t, mat)
    return jnp.concatenate(
        [_dot(t[:, i * group:(i + 1) * group], mat) for i in range(n)], axis=1)


def _noise_img(rv_ref, m_ref, eps):
    a = rv_ref[...]
    wb_n, hb_n = a.shape[0], a.shape[1]
    n = _dot(a.reshape(wb_n * hb_n, 64), m_ref[...]) * eps
    return (n.reshape(wb_n, hb_n, 8, 8).transpose(1, 2, 0, 3)
            .reshape(hb_n * 8, wb_n * 8))


def _fused_kernel(x_ref, rv0_ref, rv1_ref, rv2_ref, eps_ref,
                  mty_ref, mtc_ref, pav_ref, urow_ref, hav_ref, hup_ref,
                  o_ref):
    eps = eps_ref[0, 0]
    xr = x_ref[0]
    xg = x_ref[1]
    xb = x_ref[2]
    y = 0.299 * xr + 0.587 * xg + 0.114 * xb
    cb = -0.168736 * xr - 0.331264 * xg + 0.5 * xb + 128.0
    cr = 0.5 * xr - 0.418688 * xg - 0.081312 * xb + 128.0

    Gc = hup_ref.shape[0]

    y_o = y + _noise_img(rv0_ref, mty_ref, eps)

    def chroma(rv_ref, plane):
        n = _noise_img(rv_ref, mtc_ref, eps)
        pd = _dot(pav_ref[...], plane)
        pd = _lane_groups(pd, hav_ref[...], 2 * Gc)
        s = pd + n
        u = _dot(urow_ref[...], s)
        return _lane_groups(u, hup_ref[...], Gc)

    cb_o = chroma(rv1_ref, cb)
    cr_o = chroma(rv2_ref, cr)

    cbm = cb_o - 128.0
    crm = cr_o - 128.0
    o_ref[0] = jnp.clip(y_o + 1.402 * crm, 0.0, 255.0)
    o_ref[1] = jnp.clip(y_o - 0.344136 * cbm - 0.714136 * crm, 0.0, 255.0)
    o_ref[2] = jnp.clip(y_o + 1.772 * cbm, 0.0, 255.0)


def kernel(x, rv0, rv1, rv2, epsilon):
    B, C, H, W = x.shape
    assert C == 3 and H % 16 == 0 and W % 16 == 0
    Hb, Wb = H // 8, W // 8
    Hc, Wc = H // 2, W // 2

    R = 512
    while H % R or R > H:
        R //= 2
    Rc = R // 2

    xf = x.astype(jnp.float32)
    eps2 = epsilon.astype(jnp.float32).reshape(B, 1, 1)
    rv0i = rv0.astype(jnp.float32).reshape(B, Wb, Hb, 64)
    rv1i = rv1.astype(jnp.float32).reshape(B, Wc // 8, Hc // 8, 64)
    rv2i = rv2.astype(jnp.float32).reshape(B, Wc // 8, Hc // 8, 64)

    c = _consts(R, W)
    grid = (B, H // R)

    def const_spec(a):
        return pl.BlockSpec(a.shape, lambda b, t: (0,) * a.ndim)

    out = pl.pallas_call(
        _fused_kernel,
        out_shape=jax.ShapeDtypeStruct((B, 3, H, W), jnp.float32),
        grid=grid,
        in_specs=[
            pl.BlockSpec((None, 3, R, W), lambda b, t: (b, 0, t, 0)),
            pl.BlockSpec((None, Wb, R // 8, 64), lambda b, t: (b, 0, t, 0)),
            pl.BlockSpec((None, Wc // 8, Rc // 8, 64), lambda b, t: (b, 0, t, 0)),
            pl.BlockSpec((None, Wc // 8, Rc // 8, 64), lambda b, t: (b, 0, t, 0)),
            pl.BlockSpec((None, 1, 1), lambda b, t: (b, 0, 0)),
            const_spec(c["mty"]),
            const_spec(c["mtc"]),
            const_spec(c["pav"]),
            const_spec(c["urow"]),
            const_spec(c["hav"]),
            const_spec(c["hup"]),
        ],
        out_specs=pl.BlockSpec((None, 3, R, W), lambda b, t: (b, 0, t, 0)),
        compiler_params=pltpu.CompilerParams(
            dimension_semantics=("parallel", "parallel"),
            vmem_limit_bytes=_VMEM_LIMIT),
    )(xf, rv0i, rv1i, rv2i, eps2,
      c["mty"], c["mtc"], c["pav"], c["urow"], c["hav"], c["hup"])
    return out
